# Initial kernel scaffold; baseline (speedup 1.0000x reference)
import jax
import jax.numpy as jnp
from jax.experimental import pallas as pl


def kernel(x, edge_index, edge_vals, poly_logits, hp_alpha):
    raise NotImplementedError("write your pallas kernel here")


# speedup vs baseline: 2.8206x; 2.8206x over previous
import functools

import jax
import jax.numpy as jnp
from jax import lax
from jax.experimental import pallas as pl
from jax.experimental.pallas import tpu as pltpu
from jax.experimental.pallas import tpu_sc as plsc

_GRP = 128


@functools.lru_cache(maxsize=None)
def _build_spmm(n, h, n_pad, n_groups):
    nc, ns = 2, 16
    nw = nc * ns
    assert n_groups % nw == 0
    gpw = n_groups // nw
    rpt = n_pad // ns

    mesh = plsc.VectorSubcoreMesh(core_axis_name="c", subcore_axis_name="s")

    @functools.partial(
        pl.kernel,
        mesh=mesh,
        out_type=jax.ShapeDtypeStruct((nc, n_pad, h), jnp.float32),
        scratch_types=[
            pltpu.VMEM((_GRP,), jnp.int32),
            pltpu.VMEM((_GRP,), jnp.int32),
            pltpu.VMEM((_GRP * 16,), jnp.float32),
            pltpu.VMEM((_GRP, h), jnp.float32),
            pltpu.VMEM_SHARED((n_pad, h), jnp.float32),
            pltpu.SemaphoreType.DMA,
        ],
    )
    def spmm(x_hbm, cols_hbm, rows_hbm, vals_hbm, out_hbm,
             colv, rowv, valv, msgs, acc, sem):
        c = lax.axis_index("c")
        s = lax.axis_index("s")
        wid = s * nc + c

        zeros16 = jnp.zeros((16,), jnp.float32)

        def zero_body(i, _):
            for hh in range(h // 16):
                msgs[i, pl.ds(hh * 16, 16)] = zeros16
            return 0

        lax.fori_loop(0, _GRP, zero_body, 0)
        n_zcopy = rpt // _GRP
        assert n_zcopy * _GRP == rpt

        def zcopy_body(j, _):
            pltpu.sync_copy(msgs, acc.at[pl.ds(s * rpt + j * _GRP, _GRP)])
            return 0

        lax.fori_loop(0, n_zcopy, zcopy_body, 0)
        plsc.subcore_barrier()

        def group_body(g, _):
            grow = wid * gpw + g
            pltpu.sync_copy(cols_hbm.at[grow], colv)
            pltpu.sync_copy(rows_hbm.at[grow], rowv)
            pltpu.sync_copy(vals_hbm.at[grow], valv)
            pltpu.async_copy(x_hbm.at[colv], msgs, sem).wait()

            def scale_body(e, _):
                val16 = valv[pl.ds(e * 16, 16)]
                for hh in range(h // 16):
                    sl = pl.ds(hh * 16, 16)
                    msgs[e, sl] = msgs[e, sl] * val16
                return 0

            lax.fori_loop(0, _GRP, scale_body, 0)
            pltpu.sync_copy(msgs, acc.at[rowv], add=True)
            return 0

        lax.fori_loop(0, gpw, group_body, 0)
        plsc.subcore_barrier()

        pltpu.sync_copy(acc.at[pl.ds(s * rpt, rpt)],
                        out_hbm.at[c, pl.ds(s * rpt, rpt)])

    return spmm


def kernel(x, edge_index, edge_vals, poly_logits, hp_alpha):
    n, h = x.shape
    e = edge_index.shape[1]
    kk = poly_logits.shape[0] - 1

    nw_grp = 32 * _GRP
    e_pad = ((e + nw_grp - 1) // nw_grp) * nw_grp
    pad = e_pad - e
    n_groups = e_pad // _GRP
    n_pad = ((n + 2047) // 2048) * 2048

    rows = jnp.pad(edge_index[0], (0, pad)).reshape(n_groups, _GRP)
    cols = jnp.pad(edge_index[1], (0, pad)).reshape(n_groups, _GRP)
    vals = jnp.repeat(jnp.pad(edge_vals, (0, pad)), 16).reshape(
        n_groups, _GRP * 16)
    ones = jnp.repeat(jnp.pad(jnp.ones((e,), jnp.float32), (0, pad)),
                      16).reshape(n_groups, _GRP * 16)

    spmm = _build_spmm(n, h, n_pad, n_groups)

    def spmm_sum(v, vv):
        p = spmm(v, cols, rows, vv)
        return p[0, :n] + p[1, :n]

    deg = spmm_sum(jnp.ones((n, h), jnp.float32), ones)[:, 0]
    lam_max = 2.0 * jnp.max(deg)
    cc = 2.0 / lam_max

    w = jax.nn.softmax(poly_logits)

    s1 = spmm_sum(x, vals)
    t0 = x
    t1 = cc * s1 - x
    out = w[0] * t0 + w[1] * t1
    for k in range(1, kk):
        sk = spmm_sum(t1, vals)
        tk1 = 2.0 * (cc * sk - t1) - t0
        out = out + w[k + 1] * tk1
        t0, t1 = t1, tk1

    hp = hp_alpha * (x - s1 / lam_max)
    return -out + hp


# speedup vs baseline: 2.9869x; 1.0589x over previous
import functools

import jax
import jax.numpy as jnp
from jax import lax
from jax.experimental import pallas as pl
from jax.experimental.pallas import tpu as pltpu
from jax.experimental.pallas import tpu_sc as plsc

_GRP = 128


@functools.lru_cache(maxsize=None)
def _build_spmm(n, h, n_pad, n_groups):
    nc, ns = 2, 16
    nw = nc * ns
    assert n_groups % nw == 0
    gpw = n_groups // nw
    rpt = n_pad // ns

    assert gpw >= 4 and gpw % 2 == 0
    mesh = plsc.VectorSubcoreMesh(core_axis_name="c", subcore_axis_name="s")

    @functools.partial(
        pl.kernel,
        mesh=mesh,
        out_type=jax.ShapeDtypeStruct((nc, n_pad, h), jnp.float32),
        scratch_types=[
            pltpu.VMEM((2, 2, _GRP), jnp.int32),
            pltpu.VMEM((2, _GRP * 16,), jnp.float32),
            pltpu.VMEM((2, _GRP, h), jnp.float32),
            pltpu.VMEM_SHARED((n_pad, h), jnp.float32),
            pltpu.SemaphoreType.DMA,
            pltpu.SemaphoreType.DMA,
            pltpu.SemaphoreType.DMA,
            pltpu.SemaphoreType.DMA,
        ],
    )
    def spmm(x_hbm, cr_hbm, vals_hbm, out_hbm,
             cr, val, msgs, acc, semi0, semi1, semg0, semg1):
        c = lax.axis_index("c")
        s = lax.axis_index("s")
        wid = s * nc + c
        g0 = wid * gpw
        semi = (semi0, semi1)
        semg = (semg0, semg1)

        def fetch(g, b):
            pltpu.async_copy(cr_hbm.at[g0 + g], cr.at[b], semi[b])
            pltpu.async_copy(vals_hbm.at[g0 + g], val.at[b], semi[b])

        def fetch_wait(g, b):
            pltpu.make_async_copy(cr_hbm.at[g0 + g], cr.at[b], semi[b]).wait()
            pltpu.make_async_copy(vals_hbm.at[g0 + g], val.at[b],
                                  semi[b]).wait()

        def gather(b):
            pltpu.async_copy(x_hbm.at[cr.at[b, 0]], msgs.at[b], semg[b])

        def gather_wait(b):
            pltpu.make_async_copy(x_hbm.at[pl.ds(0, _GRP)], msgs.at[b],
                                  semg[b]).wait()

        def scale(b):
            def scale_body(e, _):
                for u in range(2):
                    val16 = val[b, pl.ds((2 * e + u) * 16, 16)]
                    for hh in range(h // 16):
                        sl = pl.ds(hh * 16, 16)
                        msgs[b, 2 * e + u, sl] = msgs[b, 2 * e + u, sl] * val16
                return 0

            lax.fori_loop(0, _GRP // 2, scale_body, 0)

        def scatter(b):
            pltpu.sync_copy(msgs.at[b], acc.at[cr.at[b, 1]], add=True)

        fetch(0, 0)

        zeros16 = jnp.zeros((16,), jnp.float32)

        def zero_body(i, _):
            for hh in range(h // 16):
                msgs[0, i, pl.ds(hh * 16, 16)] = zeros16
            return 0

        lax.fori_loop(0, _GRP, zero_body, 0)
        n_zcopy = rpt // _GRP
        assert n_zcopy * _GRP == rpt

        def zcopy_body(j, _):
            pltpu.sync_copy(msgs.at[0], acc.at[pl.ds(s * rpt + j * _GRP, _GRP)])
            return 0

        lax.fori_loop(0, n_zcopy, zcopy_body, 0)
        fetch_wait(0, 0)
        gather(0)
        fetch(1, 1)
        plsc.subcore_barrier()

        def pair_body(p, _):
            g = 2 * p
            for b in (0, 1):
                gb = g + b
                b1 = 1 - b
                fetch_wait(gb + 1, b1)
                gather(b1)
                gather_wait(b)
                scale(b)
                scatter(b)
                fetch(gb + 2, b)
            return 0

        lax.fori_loop(0, (gpw - 2) // 2, pair_body, 0)

        fetch_wait(gpw - 1, 1)
        gather(1)
        gather_wait(0)
        scale(0)
        scatter(0)
        gather_wait(1)
        scale(1)
        scatter(1)

        plsc.subcore_barrier()
        pltpu.sync_copy(acc.at[pl.ds(s * rpt, rpt)],
                        out_hbm.at[c, pl.ds(s * rpt, rpt)])

    return spmm


@functools.lru_cache(maxsize=None)
def _build_deg(n_pad, n_groups):
    nc, ns = 2, 16
    nw = nc * ns
    gpw = n_groups // nw
    rpt = n_pad // ns
    hh = 16

    mesh = plsc.VectorSubcoreMesh(core_axis_name="c", subcore_axis_name="s")

    @functools.partial(
        pl.kernel,
        mesh=mesh,
        out_type=jax.ShapeDtypeStruct((nc, n_pad, hh), jnp.float32),
        scratch_types=[
            pltpu.VMEM((2, _GRP), jnp.int32),
            pltpu.VMEM((_GRP, hh), jnp.float32),
            pltpu.VMEM_SHARED((n_pad, hh), jnp.float32),
        ],
    )
    def deg(rows_hbm, mask_hbm, out_hbm, rowv, msgs, acc):
        c = lax.axis_index("c")
        s = lax.axis_index("s")
        wid = s * nc + c
        g0 = wid * gpw

        zeros16 = jnp.zeros((16,), jnp.float32)

        def zero_body(i, _):
            msgs[i, pl.ds(0, 16)] = zeros16
            return 0

        lax.fori_loop(0, _GRP, zero_body, 0)

        def zcopy_body(j, _):
            pltpu.sync_copy(msgs, acc.at[pl.ds(s * rpt + j * _GRP, _GRP)])
            return 0

        lax.fori_loop(0, rpt // _GRP, zcopy_body, 0)
        plsc.subcore_barrier()

        def group_body(g, _):
            pltpu.sync_copy(rows_hbm.at[g0 + g], rowv.at[0])
            pltpu.sync_copy(mask_hbm.at[g0 + g], msgs)
            pltpu.sync_copy(msgs, acc.at[rowv.at[0]], add=True)
            return 0

        lax.fori_loop(0, gpw, group_body, 0)
        plsc.subcore_barrier()
        pltpu.sync_copy(acc.at[pl.ds(s * rpt, rpt)],
                        out_hbm.at[c, pl.ds(s * rpt, rpt)])

    return deg


def kernel(x, edge_index, edge_vals, poly_logits, hp_alpha):
    n, h = x.shape
    e = edge_index.shape[1]
    kk = poly_logits.shape[0] - 1

    nw_grp = 32 * _GRP * 2
    e_pad = ((e + nw_grp - 1) // nw_grp) * nw_grp
    pad = e_pad - e
    n_groups = e_pad // _GRP
    n_pad = ((n + 2047) // 2048) * 2048

    rows = jnp.pad(edge_index[0], (0, pad)).reshape(n_groups, 1, _GRP)
    cols = jnp.pad(edge_index[1], (0, pad)).reshape(n_groups, 1, _GRP)
    cr = jnp.concatenate([cols, rows], axis=1)
    vals = jnp.repeat(jnp.pad(edge_vals, (0, pad)), 16).reshape(
        n_groups, _GRP * 16)
    ones = jnp.repeat(jnp.pad(jnp.ones((e,), jnp.float32), (0, pad)),
                      16).reshape(n_groups, _GRP * 16)

    spmm = _build_spmm(n, h, n_pad, n_groups)

    def spmm_sum(v, vv):
        p = spmm(v, cr, vv)
        return p[0, :n] + p[1, :n]

    deg = spmm_sum(jnp.ones((n, h), jnp.float32), ones)[:, 0]
    lam_max = 2.0 * jnp.max(deg)
    cc = 2.0 / lam_max

    w = jax.nn.softmax(poly_logits)

    s1 = spmm_sum(x, vals)
    t0 = x
    t1 = cc * s1 - x
    out = w[0] * t0 + w[1] * t1
    for k in range(1, kk):
        sk = spmm_sum(t1, vals)
        tk1 = 2.0 * (cc * sk - t1) - t0
        out = out + w[k + 1] * tk1
        t0, t1 = t1, tk1

    hp = hp_alpha * (x - s1 / lam_max)
    return -out + hp
